# baseline (speedup 1.0000x reference)
import jax
import jax.numpy as jnp
from jax.experimental import pallas as pl


def _cheb_j(x, row, col, ew, Ws, b):
    n = x.shape[0]
    deg = jax.ops.segment_sum(ew, row, num_segments=n)
    dinv = jnp.where(deg > 0, 1.0 / jnp.sqrt(jnp.maximum(deg, 1e-12)), 0.0)
    nw = -dinv[row] * ew * dinv[col]

    def prop(h):
        return jax.ops.segment_sum(nw[:, None] * h[row], col, num_segments=n)

    Tx0 = x
    out = Tx0 @ Ws[0]
    Tx1 = None
    if len(Ws) > 1:
        Tx1 = prop(Tx0)
        out = out + Tx1 @ Ws[1]
    for k in range(2, len(Ws)):
        Tx2 = 2.0 * prop(Tx1) - Tx0
        out = out + Tx2 @ Ws[k]
        Tx0, Tx1 = Tx1, Tx2
    return out + b


def _bn_j(h, g, b):
    m = jnp.mean(h, axis=0)
    v = jnp.mean((h - m) ** 2, axis=0)
    return (h - m) / jnp.sqrt(v + 1e-5) * g + b


def _logsoftmax_kernel(h_ref, o_ref):
    h = h_ref[...]
    m = jnp.max(h, axis=1, keepdims=True)
    e = jnp.exp(h - m)
    s = jnp.sum(e, axis=1, keepdims=True)
    o_ref[...] = (h - m) - jnp.log(s)


def kernel(x, edge_index, edge_weight, W1_0, b1, W2_0, W2_1, b2, W3_0, W3_1, W3_2, b3, Wfc, bfc, g1, be1, g2, be2, g3, be3):
    row = edge_index[0]
    col = edge_index[1]
    h = jax.nn.relu(_bn_j(_cheb_j(x, row, col, edge_weight, [W1_0], b1), g1, be1))
    h = jax.nn.relu(_bn_j(_cheb_j(h, row, col, edge_weight, [W2_0, W2_1], b2), g2, be2))
    h = _bn_j(_cheb_j(h, row, col, edge_weight, [W3_0, W3_1, W3_2], b3), g3, be3)
    h = h @ Wfc + bfc
    return pl.pallas_call(
        _logsoftmax_kernel,
        out_shape=jax.ShapeDtypeStruct(h.shape, h.dtype),
    )(h)


# speedup vs baseline: 4.4049x; 4.4049x over previous
import functools

import jax
import jax.numpy as jnp
from jax import lax
from jax.experimental import pallas as pl
from jax.experimental.pallas import tpu as pltpu
from jax.experimental.pallas import tpu_sc as plsc

N = 10000
D = 128
E = 320000
NPAD = 10240
EPAD = 323584
CH = 128
RB = 400
NT = 16
RPT = NPAD // NT

_mesh = plsc.VectorSubcoreMesh(core_axis_name="c", subcore_axis_name="s")


def _deg_body(row_hbm, ew_hbm, out_hbm, rowb, ewb, zbuf, acc):
    c = lax.axis_index("c")
    s = lax.axis_index("s")

    def zrow(r, _):
        zbuf[pl.ds(r * 16, 16)] = jnp.zeros((16,), jnp.float32)
        return 0

    lax.fori_loop(0, RPT // 16, zrow, 0)
    pltpu.sync_copy(zbuf, acc.at[pl.ds(s * RPT, RPT)])
    plsc.subcore_barrier()

    base = c * (EPAD // 2) + s * (EPAD // 32)

    def chunk(k, _):
        off = base + k * CH
        pltpu.sync_copy(row_hbm.at[pl.ds(off, CH)], rowb.at[0])
        pltpu.sync_copy(ew_hbm.at[pl.ds(off, CH)], ewb)
        pltpu.sync_copy(ewb, acc.at[rowb.at[0]], add=True)
        return 0

    lax.fori_loop(0, EPAD // 32 // CH, chunk, 0)
    plsc.subcore_barrier()
    pltpu.sync_copy(acc.at[pl.ds(s * RPT, RPT)], out_hbm.at[c, pl.ds(s * RPT, RPT)])


def _deg_call(rowp, ewp):
    return pl.kernel(
        _deg_body,
        out_type=jax.ShapeDtypeStruct((2, NPAD), jnp.float32),
        mesh=_mesh,
        scratch_types=[
            pltpu.VMEM((1, CH), jnp.int32),
            pltpu.VMEM((CH,), jnp.float32),
            pltpu.VMEM((RPT,), jnp.float32),
            pltpu.VMEM_SHARED((NPAD,), jnp.float32),
        ],
    )(rowp, ewp)


def _prop_body(noff, tab_hbm, row_hbm, col_hbm, ew_hbm, dinv_hbm, out_hbm,
               rowb, colb, rowg, ewb, drb, dcb, nwb, gath, zbuf, acc):
    c = lax.axis_index("c")
    s = lax.axis_index("s")

    def zrow(r, _):
        for j in range(8):
            zbuf[r, pl.ds(j * 16, 16)] = jnp.zeros((16,), jnp.float32)
        return 0

    lax.fori_loop(0, 64, zrow, 0)
    for t in range(RPT // 64):
        pltpu.sync_copy(zbuf, acc.at[pl.ds(s * RPT + t * 64, 64)])
    plsc.subcore_barrier()

    if noff:
        base = s * (EPAD // 16)
        nchunks = EPAD // 16 // CH
        ioff = c * noff
    else:
        base = c * (EPAD // 2) + s * (EPAD // 32)
        nchunks = EPAD // 32 // CH
        ioff = 0

    def chunk(k, _):
        off = base + k * CH
        pltpu.sync_copy(row_hbm.at[pl.ds(off, CH)], rowb.at[0])
        pltpu.sync_copy(col_hbm.at[pl.ds(off, CH)], colb.at[0])
        pltpu.sync_copy(ew_hbm.at[pl.ds(off, CH)], ewb)
        pltpu.sync_copy(dinv_hbm.at[rowb.at[0]], drb)
        pltpu.sync_copy(dinv_hbm.at[colb.at[0]], dcb)
        for i in range(CH // 16):
            sl = pl.ds(i * 16, 16)
            nwb[sl] = -(drb[sl] * ewb[sl] * dcb[sl])
            rowg[0, sl] = rowb[0, sl] + ioff
        pltpu.sync_copy(tab_hbm.at[rowg.at[0]], gath)

        def group(gi, _):
            nw16 = nwb[pl.ds(gi * 16, 16)]
            for l in range(16):
                w = nw16[l]
                e = gi * 16 + l
                for j in range(8):
                    sj = pl.ds(j * 16, 16)
                    gath[e, sj] = gath[e, sj] * w
            return 0

        lax.fori_loop(0, CH // 16, group, 0)
        pltpu.sync_copy(gath, acc.at[colb.at[0]], add=True)
        return 0

    lax.fori_loop(0, nchunks, chunk, 0)
    plsc.subcore_barrier()
    pltpu.sync_copy(acc.at[pl.ds(s * RPT, RPT)], out_hbm.at[c, pl.ds(s * RPT, RPT)])


def _prop_call(tab, rowp, colp, ewp, dinv, noff):
    return pl.kernel(
        functools.partial(_prop_body, noff),
        out_type=jax.ShapeDtypeStruct((2, NPAD, D), jnp.float32),
        mesh=_mesh,
        scratch_types=[
            pltpu.VMEM((1, CH), jnp.int32),
            pltpu.VMEM((1, CH), jnp.int32),
            pltpu.VMEM((1, CH), jnp.int32),
            pltpu.VMEM((CH,), jnp.float32),
            pltpu.VMEM((CH,), jnp.float32),
            pltpu.VMEM((CH,), jnp.float32),
            pltpu.VMEM((CH,), jnp.float32),
            pltpu.VMEM((CH, D), jnp.float32),
            pltpu.VMEM((64, D), jnp.float32),
            pltpu.VMEM_SHARED((NPAD, D), jnp.float32),
        ],
    )(tab, rowp, colp, ewp, dinv)


def _dinv_kernel(dp_ref, o_ref):
    deg = dp_ref[0] + dp_ref[1]
    o_ref[...] = jnp.where(
        deg > 0, lax.rsqrt(jnp.maximum(deg, 1e-12)), 0.0)


def _stats_update(st_ref, y, j):
    s = jnp.sum(y, axis=0, keepdims=True)
    ss = jnp.sum(y * y, axis=0, keepdims=True)
    upd = jnp.concatenate(
        [s, ss, jnp.zeros((6, y.shape[1]), jnp.float32)], axis=0)

    @pl.when(j == 0)
    def _():
        st_ref[...] = upd

    @pl.when(j > 0)
    def _():
        st_ref[...] = st_ref[...] + upd


def _mm_stats_kernel(a_ref, w_ref, b_ref, y_ref, st_ref):
    j = pl.program_id(0)
    y = jnp.dot(a_ref[...], w_ref[...], preferred_element_type=jnp.float32,
                precision=lax.Precision.HIGHEST) + b_ref[...]
    y_ref[...] = y
    _stats_update(st_ref, y, j)


def _bn_from_stats(y, st, g, be):
    m = st[0:1, :] * (1.0 / N)
    v = st[1:2, :] * (1.0 / N) - m * m
    r = lax.rsqrt(v + 1e-5)
    return (y - m) * r * g + be


def _norm_mm2_kernel(y_ref, st_ref, g_ref, be_ref, w0_ref, w1_ref,
                     g2_ref, p2_ref):
    h = jnp.maximum(_bn_from_stats(y_ref[...], st_ref[...], g_ref[...],
                                   be_ref[...]), 0.0)
    g2_ref[...] = jnp.dot(h, w0_ref[...], preferred_element_type=jnp.float32,
                          precision=lax.Precision.HIGHEST)
    p = jnp.dot(h, w1_ref[...], preferred_element_type=jnp.float32,
                precision=lax.Precision.HIGHEST)
    p2_ref[0] = p[:, :D]
    p2_ref[1] = p[:, D:]


def _sum_stats2_kernel(g2_ref, pa_ref, pb_ref, b_ref, y_ref, st_ref):
    j = pl.program_id(0)
    y = g2_ref[...] + jnp.concatenate([pa_ref[...], pb_ref[...]], axis=1) \
        + b_ref[...]
    y_ref[...] = y
    _stats_update(st_ref, y, j)


def _norm_mm3_kernel(y_ref, st_ref, g_ref, be_ref, w0_ref, w1_ref, w2_ref,
                     a3_ref, bc_ref):
    h = jnp.maximum(_bn_from_stats(y_ref[...], st_ref[...], g_ref[...],
                                   be_ref[...]), 0.0)
    a3_ref[...] = jnp.dot(h, w0_ref[...] - w2_ref[...],
                          preferred_element_type=jnp.float32,
                          precision=lax.Precision.HIGHEST)
    bc_ref[0] = jnp.dot(h, w1_ref[...], preferred_element_type=jnp.float32,
                        precision=lax.Precision.HIGHEST)
    bc_ref[1] = jnp.dot(h, w2_ref[...], preferred_element_type=jnp.float32,
                        precision=lax.Precision.HIGHEST)


def _sum_stats3_kernel(a3_ref, pb_ref, q0_ref, q1_ref, b_ref, y_ref, st_ref):
    j = pl.program_id(0)
    y = a3_ref[...] + pb_ref[...] + 2.0 * (q0_ref[...] + q1_ref[...]) \
        + b_ref[...]
    y_ref[...] = y
    _stats_update(st_ref, y, j)


def _final_kernel(y_ref, st_ref, g_ref, be_ref, wfc_ref, bfc_ref, o_ref):
    h = _bn_from_stats(y_ref[...], st_ref[...], g_ref[...], be_ref[...])
    z = jnp.dot(h, wfc_ref[...], preferred_element_type=jnp.float32,
                precision=lax.Precision.HIGHEST) + bfc_ref[...]
    m = jnp.max(z, axis=1, keepdims=True)
    e = jnp.exp(z - m)
    l = jnp.log(jnp.sum(e, axis=1, keepdims=True))
    o_ref[...] = z - m - l


def _whole(shape):
    return pl.BlockSpec(shape, lambda j: (0,) * len(shape))


def _rows(cols):
    return pl.BlockSpec((RB, cols), lambda j: (j, 0))


_G = N // RB


def kernel(x, edge_index, edge_weight, W1_0, b1, W2_0, W2_1, b2, W3_0, W3_1,
           W3_2, b3, Wfc, bfc, g1, be1, g2, be2, g3, be3):
    f32 = jnp.float32
    pad = EPAD - E
    rowp = jnp.concatenate([edge_index[0], jnp.zeros((pad,), jnp.int32)])
    colp = jnp.concatenate([edge_index[1], jnp.zeros((pad,), jnp.int32)])
    ewp = jnp.concatenate([edge_weight, jnp.zeros((pad,), f32)])

    degp = _deg_call(rowp, ewp)
    dinv = pl.pallas_call(
        _dinv_kernel,
        out_shape=jax.ShapeDtypeStruct((NPAD // 128, 128), f32),
    )(degp.reshape(2, NPAD // 128, 128)).reshape(NPAD)

    Y1, st1 = pl.pallas_call(
        _mm_stats_kernel,
        grid=(_G,),
        in_specs=[_rows(D), _whole((D, 512)), _whole((1, 512))],
        out_specs=[_rows(512), _whole((8, 512))],
        out_shape=[jax.ShapeDtypeStruct((N, 512), f32),
                   jax.ShapeDtypeStruct((8, 512), f32)],
    )(x, W1_0, b1.reshape(1, 512))

    G2, P2 = pl.pallas_call(
        _norm_mm2_kernel,
        grid=(_G,),
        in_specs=[_rows(512), _whole((8, 512)), _whole((1, 512)),
                  _whole((1, 512)), _whole((512, 256)), _whole((512, 256))],
        out_specs=[_rows(256), pl.BlockSpec((2, RB, D), lambda j: (0, j, 0))],
        out_shape=[jax.ShapeDtypeStruct((N, 256), f32),
                   jax.ShapeDtypeStruct((2, N, D), f32)],
    )(Y1, st1, g1.reshape(1, 512), be1.reshape(1, 512), W2_0, W2_1)

    prop1 = _prop_call(P2.reshape(2 * N, D), rowp, colp, ewp, dinv, N)

    Y2, st2 = pl.pallas_call(
        _sum_stats2_kernel,
        grid=(_G,),
        in_specs=[_rows(256), _rows(D), _rows(D), _whole((1, 256))],
        out_specs=[_rows(256), _whole((8, 256))],
        out_shape=[jax.ShapeDtypeStruct((N, 256), f32),
                   jax.ShapeDtypeStruct((8, 256), f32)],
    )(G2, prop1[0, :N], prop1[1, :N], b2.reshape(1, 256))

    A3, BC = pl.pallas_call(
        _norm_mm3_kernel,
        grid=(_G,),
        in_specs=[_rows(256), _whole((8, 256)), _whole((1, 256)),
                  _whole((1, 256)), _whole((256, D)), _whole((256, D)),
                  _whole((256, D))],
        out_specs=[_rows(D), pl.BlockSpec((2, RB, D), lambda j: (0, j, 0))],
        out_shape=[jax.ShapeDtypeStruct((N, D), f32),
                   jax.ShapeDtypeStruct((2, N, D), f32)],
    )(Y2, st2, g2.reshape(1, 256), be2.reshape(1, 256), W3_0, W3_1, W3_2)

    prop2 = _prop_call(BC.reshape(2 * N, D), rowp, colp, ewp, dinv, N)
    prop3 = _prop_call(prop2[1, :N], rowp, colp, ewp, dinv, 0)

    Y3, st3 = pl.pallas_call(
        _sum_stats3_kernel,
        grid=(_G,),
        in_specs=[_rows(D), _rows(D), _rows(D), _rows(D), _whole((1, D))],
        out_specs=[_rows(D), _whole((8, D))],
        out_shape=[jax.ShapeDtypeStruct((N, D), f32),
                   jax.ShapeDtypeStruct((8, D), f32)],
    )(A3, prop2[0, :N], prop3[0, :N], prop3[1, :N], b3.reshape(1, D))

    out = pl.pallas_call(
        _final_kernel,
        grid=(_G,),
        in_specs=[_rows(D), _whole((8, D)), _whole((1, D)), _whole((1, D)),
                  _whole((D, 6)), _whole((1, 6))],
        out_specs=_rows(6),
        out_shape=jax.ShapeDtypeStruct((N, 6), f32),
    )(Y3, st3, g3.reshape(1, D), be3.reshape(1, D), Wfc, bfc.reshape(1, 6))
    return out
